# baseline (speedup 1.0000x reference)
import functools

import jax
import jax.numpy as jnp
from jax import lax
from jax.experimental import pallas as pl
from jax.experimental.pallas import tpu as pltpu
from jax.experimental.pallas import tpu_sc as plsc

_NUM_CORES = 2
_NUM_SUBCORES = 16
_NUM_WORKERS = _NUM_CORES * _NUM_SUBCORES


@functools.lru_cache(maxsize=None)
def _make_embed(batch, ctx, dim, blk):
    mesh = plsc.VectorSubcoreMesh(
        core_axis_name="c",
        subcore_axis_name="s",
        num_cores=_NUM_CORES,
        num_subcores=_NUM_SUBCORES,
    )

    @functools.partial(
        pl.kernel,
        out_type=jax.ShapeDtypeStruct((batch * ctx, dim), jnp.float32),
        mesh=mesh,
        scratch_types=[
            pltpu.VMEM((batch, blk), jnp.int32),
            pltpu.VMEM((blk, dim), jnp.float32),
            pltpu.VMEM((batch * blk, dim), jnp.float32),
            pltpu.SemaphoreType.DMA,
            pltpu.SemaphoreType.DMA,
            pltpu.SemaphoreType.DMA((batch,)),
            pltpu.SemaphoreType.DMA((batch,)),
            pltpu.SemaphoreType.DMA((batch,)),
        ],
    )
    def body(tok_hbm, table_hbm, pos_hbm, out_hbm, idx_v, pos_v, acc_v,
             s_idx, s_pos, s_rep, s_g, s_o):
        wid = lax.axis_index("s") * _NUM_CORES + lax.axis_index("c")
        col0 = wid * blk

        idx_cp = [
            pltpu.async_copy(
                tok_hbm.at[b, pl.ds(col0, blk)], idx_v.at[b], s_idx)
            for b in range(batch)
        ]
        pos_cp = pltpu.async_copy(
            pos_hbm.at[pl.ds(col0, blk)], pos_v, s_pos)
        pos_cp.wait()
        for c in idx_cp:
            c.wait()

        lanes = 16
        n_ch = dim // lanes

        def rep_quarter(b):
            def rep_row(i, carry):
                for c in range(n_ch):
                    sl = pl.ds(c * lanes, lanes)
                    acc_v[b * blk + i, sl] = pos_v[i, sl]
                return carry
            lax.fori_loop(0, blk, rep_row, 0, unroll=4)

        g_cp = []
        for b in range(batch):
            rep_quarter(b)
            g_cp.append(pltpu.async_copy(
                table_hbm.at[idx_v.at[b]],
                acc_v.at[pl.ds(b * blk, blk)], s_g.at[b], add=True))
        o_cp = []
        for b in range(batch):
            g_cp[b].wait()
            o_cp.append(pltpu.async_copy(
                acc_v.at[pl.ds(b * blk, blk)],
                out_hbm.at[pl.ds(b * ctx + col0, blk)], s_o.at[b]))
        for c in o_cp:
            c.wait()

    return body


def kernel(token_ids, word_embeddings, pos_embeddings):
    batch, ctx = token_ids.shape
    _, dim = word_embeddings.shape
    blk = ctx // _NUM_WORKERS
    fn = _make_embed(batch, ctx, dim, blk)
    out = fn(token_ids.astype(jnp.int32), word_embeddings.astype(jnp.float32),
             pos_embeddings.astype(jnp.float32))
    return out.reshape(batch, ctx, dim)


# speedup vs baseline: 1.1336x; 1.1336x over previous
import functools

import jax
import jax.numpy as jnp
from jax import lax
from jax.experimental import pallas as pl
from jax.experimental.pallas import tpu as pltpu
from jax.experimental.pallas import tpu_sc as plsc

_NUM_CORES = 2
_NUM_SUBCORES = 16
_NUM_WORKERS = _NUM_CORES * _NUM_SUBCORES


@functools.lru_cache(maxsize=None)
def _make_embed(batch, ctx, dim, blk):
    mesh = plsc.VectorSubcoreMesh(
        core_axis_name="c",
        subcore_axis_name="s",
        num_cores=_NUM_CORES,
        num_subcores=_NUM_SUBCORES,
    )

    @functools.partial(
        pl.kernel,
        out_type=jax.ShapeDtypeStruct((batch * ctx, dim), jnp.float32),
        mesh=mesh,
        scratch_types=[
            pltpu.VMEM((batch, blk), jnp.int32),
            pltpu.VMEM_SHARED((_NUM_SUBCORES * blk, dim), jnp.float32),
            pltpu.VMEM((batch * blk, dim), jnp.float32),
            pltpu.SemaphoreType.DMA,
            pltpu.SemaphoreType.DMA,
            pltpu.SemaphoreType.DMA((batch,)),
            pltpu.SemaphoreType.DMA((batch,)),
            pltpu.SemaphoreType.DMA((batch,)),
        ],
    )
    def body(tok_hbm, table_hbm, pos_hbm, out_hbm, idx_v, spos, acc_v,
             s_idx, s_pos, s_rep, s_g, s_o):
        sid = lax.axis_index("s")
        wid = sid * _NUM_CORES + lax.axis_index("c")
        col0 = wid * blk
        my_spos = spos.at[pl.ds(sid * blk, blk)]

        idx_cp = [
            pltpu.async_copy(
                tok_hbm.at[b, pl.ds(col0, blk)], idx_v.at[b], s_idx)
            for b in range(batch)
        ]
        pos_cp = pltpu.async_copy(
            pos_hbm.at[pl.ds(col0, blk)], my_spos, s_pos)
        pos_cp.wait()
        rep_cp = [
            pltpu.async_copy(my_spos, acc_v.at[pl.ds(b * blk, blk)],
                             s_rep.at[b])
            for b in range(batch)
        ]
        for c in idx_cp:
            c.wait()
        g_cp = []
        for b in range(batch):
            rep_cp[b].wait()
            g_cp.append(pltpu.async_copy(
                table_hbm.at[idx_v.at[b]],
                acc_v.at[pl.ds(b * blk, blk)], s_g.at[b], add=True))
        o_cp = []
        for b in range(batch):
            g_cp[b].wait()
            o_cp.append(pltpu.async_copy(
                acc_v.at[pl.ds(b * blk, blk)],
                out_hbm.at[pl.ds(b * ctx + col0, blk)], s_o.at[b]))
        for c in o_cp:
            c.wait()

    return body


def kernel(token_ids, word_embeddings, pos_embeddings):
    batch, ctx = token_ids.shape
    _, dim = word_embeddings.shape
    blk = ctx // _NUM_WORKERS
    fn = _make_embed(batch, ctx, dim, blk)
    out = fn(token_ids.astype(jnp.int32), word_embeddings.astype(jnp.float32),
             pos_embeddings.astype(jnp.float32))
    return out.reshape(batch, ctx, dim)
